# baseline (speedup 1.0000x reference)
import functools

import jax
import jax.numpy as jnp
from jax import lax
from jax.experimental import pallas as pl
from jax.experimental.pallas import tpu as pltpu
from jax.experimental.pallas import tpu_sc as plsc

MAXLEN = 200
EMBED = 32
LANES = 16
NWORKERS = 32
GATHER_ROWS = 100
GATHERS_PER_CHUNK = 16
CHUNK = GATHER_ROWS * GATHERS_PER_CHUNK

VOCAB = 1000000
LIN_BLOCK_V = 2048


def _linearize_table(tok_t):
    grid = (VOCAB + LIN_BLOCK_V - 1) // LIN_BLOCK_V
    rows_out = LIN_BLOCK_V // 4

    def body(in_ref, out_ref):
        x = in_ref[...]
        z = jnp.concatenate([x, x, x, x], axis=0)
        zt = z.T.reshape(rows_out, 4, 128)
        lane = lax.broadcasted_iota(jnp.int32, (rows_out, 128), 1)
        q = lane // EMBED
        out = zt[:, 0, :]
        for k in range(1, 4):
            out = jnp.where(q == k, zt[:, k, :], out)
        out_ref[...] = out

    return pl.pallas_call(
        body,
        grid=(grid,),
        in_specs=[pl.BlockSpec((EMBED, LIN_BLOCK_V), lambda i: (0, i))],
        out_specs=pl.BlockSpec((rows_out, 128), lambda i: (i, 0)),
        out_shape=jax.ShapeDtypeStruct((VOCAB * EMBED // 128, 128), jnp.float32),
    )(tok_t)


VPAD = 1000448
RETILE_BLOCK = 8192

def _retile_table(tok_t):
    grid = (EMBED // 8, (VOCAB + RETILE_BLOCK - 1) // RETILE_BLOCK)

    def body(in_ref, out_ref):
        out_ref[...] = in_ref[...].reshape(8, RETILE_BLOCK // 128, 128)

    return pl.pallas_call(
        body,
        grid=grid,
        in_specs=[pl.BlockSpec((8, RETILE_BLOCK), lambda s, j: (s, j))],
        out_specs=pl.BlockSpec(
            (8, RETILE_BLOCK // 128, 128), lambda s, j: (s, j, 0)
        ),
        out_shape=jax.ShapeDtypeStruct((EMBED, VPAD // 128, 128), jnp.float32),
    )(tok_t)


F_CHUNK = 1600
F_NCHUNKS = VOCAB // F_CHUNK


def _make_fold_kernel():
    mesh = plsc.VectorSubcoreMesh(core_axis_name="c", subcore_axis_name="s")
    max_chunks = (F_NCHUNKS + NWORKERS - 1) // NWORKERS

    @functools.partial(
        pl.kernel,
        mesh=mesh,
        compiler_params=pltpu.CompilerParams(
            use_tc_tiling_on_sc=False, needs_layout_passes=False
        ),
        out_type=jax.ShapeDtypeStruct((VOCAB, EMBED), jnp.float32),
        scratch_types=[
            pltpu.VMEM((EMBED * F_CHUNK,), jnp.float32),
            pltpu.VMEM((F_CHUNK, EMBED), jnp.float32),
            pltpu.SemaphoreType.DMA,
        ],
    )
    def fold_kernel(src_hbm, out_hbm, inbuf, outbuf, sem):
        wid = lax.axis_index("s") * 2 + lax.axis_index("c")
        base_lo = lax.iota(jnp.int32, 16) * F_CHUNK
        base_hi = base_lo + 16 * F_CHUNK

        def g_body(k, carry):
            c = wid + NWORKERS * k

            @pl.when(c < F_NCHUNKS)
            def _():
                v0 = c * F_CHUNK
                cps = [
                    pltpu.async_copy(
                        src_hbm.at[pl.ds(d * VPAD + v0, F_CHUNK)],
                        inbuf.at[pl.ds(d * F_CHUNK, F_CHUNK)],
                        sem,
                    )
                    for d in range(EMBED)
                ]
                for cp in cps:
                    cp.wait()

                def i_body(i, carry2):
                    outbuf[i, pl.ds(0, 16)] = plsc.load_gather(
                        inbuf, [base_lo + i]
                    )
                    outbuf[i, pl.ds(16, 16)] = plsc.load_gather(
                        inbuf, [base_hi + i]
                    )
                    return carry2

                lax.fori_loop(0, F_CHUNK, i_body, 0)
                pltpu.sync_copy(outbuf, out_hbm.at[pl.ds(v0, F_CHUNK)])

            return carry

        lax.fori_loop(0, max_chunks, g_body, 0)

    return fold_kernel


OUT_BLOCK_B = 128


def _transpose_out(rows128, batch):
    per_b128 = MAXLEN * EMBED // 128
    grid = batch // OUT_BLOCK_B
    rows_in = OUT_BLOCK_B * per_b128

    def body(in_ref, out_ref):
        blk = in_ref[...].reshape(per_b128, OUT_BLOCK_B, 128)
        y = jnp.transpose(blk, (0, 2, 1))
        out_ref[...] = y.reshape(MAXLEN, EMBED, OUT_BLOCK_B)

    return pl.pallas_call(
        body,
        grid=(grid,),
        in_specs=[pl.BlockSpec((rows_in, 128), lambda i: (i, 0))],
        out_specs=pl.BlockSpec((MAXLEN, EMBED, OUT_BLOCK_B), lambda i: (0, 0, i)),
        out_shape=jax.ShapeDtypeStruct((MAXLEN, EMBED, batch), jnp.float32),
    )(rows128)


@functools.cache
def _make_gather_kernel(n_rows: int):
    rows_per_w = n_rows // NWORKERS
    n_chunks = rows_per_w // CHUNK
    idx_rows_per_w = rows_per_w // GATHER_ROWS
    cycles = CHUNK // MAXLEN

    mesh = plsc.VectorSubcoreMesh(core_axis_name="c", subcore_axis_name="s")
    jrows = MAXLEN * EMBED // 128
    wblock = 128 * MAXLEN * EMBED

    @functools.partial(
        pl.kernel,
        mesh=mesh,
        compiler_params=pltpu.CompilerParams(use_tc_tiling_on_sc=False),
        out_type=jax.ShapeDtypeStruct((n_rows * EMBED,), jnp.float32),
        scratch_types=[
            pltpu.VMEM((GATHERS_PER_CHUNK, GATHER_ROWS), jnp.int32),
            pltpu.VMEM((CHUNK, EMBED), jnp.float32),
            pltpu.VMEM((CHUNK * EMBED,), jnp.float32),
            pltpu.VMEM((MAXLEN, EMBED), jnp.float32),
            pltpu.SemaphoreType.DMA,
        ],
    )
    def emb_kernel(x_hbm, tok_hbm, pos_hbm, out_hbm, idx_v, rows_v, stage_v,
                   pos_v, sem):
        wid = lax.axis_index("s") * 2 + lax.axis_index("c")
        pltpu.sync_copy(pos_hbm, pos_v)

        def chunk_body(g, carry):
            r0 = wid * idx_rows_per_w + g * GATHERS_PER_CHUNK
            pltpu.sync_copy(x_hbm.at[pl.ds(r0, GATHERS_PER_CHUNK)], idx_v)
            copies = [
                pltpu.async_copy(
                    tok_hbm.at[idx_v.at[j]],
                    rows_v.at[pl.ds(j * GATHER_ROWS, GATHER_ROWS)],
                    sem,
                )
                for j in range(GATHERS_PER_CHUNK)
            ]
            for c in copies:
                c.wait()

            def t_body(t, carry2):
                p0 = pos_v[t, pl.ds(0, LANES)]
                p1 = pos_v[t, pl.ds(LANES, LANES)]
                off0 = (t // 4) * (128 * cycles) + (t % 4) * EMBED
                for cyc in range(cycles):
                    r = cyc * MAXLEN + t
                    o = off0 + cyc * 128
                    stage_v[pl.ds(o, LANES)] = rows_v[r, pl.ds(0, LANES)] + p0
                    stage_v[pl.ds(o + LANES, LANES)] = (
                        rows_v[r, pl.ds(LANES, LANES)] + p1
                    )
                return carry2

            lax.fori_loop(0, MAXLEN, t_body, 0)

            wbase = wid * wblock + g * (cycles * 128)
            outs = [
                pltpu.async_copy(
                    stage_v.at[pl.ds(j * (cycles * 128), cycles * 128)],
                    out_hbm.at[pl.ds(wbase + j * (128 * 128), cycles * 128)],
                    sem,
                )
                for j in range(jrows)
            ]
            for c in outs:
                c.wait()
            return carry

        lax.fori_loop(0, n_chunks, chunk_body, 0)

    return emb_kernel


def kernel(x, token_emb, pos_emb):
    batch, maxlen = x.shape
    n_rows = batch * maxlen
    x2 = x.reshape(n_rows // GATHER_ROWS, GATHER_ROWS).astype(jnp.int32)
    tok_lin = _linearize_table(token_emb.T).reshape(VOCAB, EMBED)
    rows = _make_gather_kernel(n_rows)(x2, tok_lin, pos_emb)
    out_tdb = _transpose_out(rows.reshape(n_rows * EMBED // 128, 128), batch)
    return jnp.transpose(out_tdb, (2, 0, 1))


# speedup vs baseline: 1.0005x; 1.0005x over previous
import functools

import jax
import jax.numpy as jnp
from jax import lax
from jax.experimental import pallas as pl
from jax.experimental.pallas import tpu as pltpu
from jax.experimental.pallas import tpu_sc as plsc

MAXLEN = 200
EMBED = 32
LANES = 16
NWORKERS = 32
GATHER_ROWS = 100
GATHERS_PER_CHUNK = 16
CHUNK = GATHER_ROWS * GATHERS_PER_CHUNK

VOCAB = 1000000
LIN_BLOCK_V = 2048


def _linearize_table(tok_t):
    grid = (VOCAB + LIN_BLOCK_V - 1) // LIN_BLOCK_V
    rows_out = LIN_BLOCK_V // 4

    def body(in_ref, out_ref):
        x = in_ref[...]
        z = jnp.concatenate([x, x, x, x], axis=0)
        zt = z.T.reshape(rows_out, 4, 128)
        lane = lax.broadcasted_iota(jnp.int32, (rows_out, 128), 1)
        q = lane // EMBED
        out = zt[:, 0, :]
        for k in range(1, 4):
            out = jnp.where(q == k, zt[:, k, :], out)
        out_ref[...] = out

    return pl.pallas_call(
        body,
        grid=(grid,),
        in_specs=[pl.BlockSpec((EMBED, LIN_BLOCK_V), lambda i: (0, i))],
        out_specs=pl.BlockSpec((rows_out, 128), lambda i: (i, 0)),
        out_shape=jax.ShapeDtypeStruct((VOCAB * EMBED // 128, 128), jnp.float32),
    )(tok_t)


VPAD = 1000448
RETILE_BLOCK = 8192

def _retile_table(tok_t):
    grid = (EMBED // 8, (VOCAB + RETILE_BLOCK - 1) // RETILE_BLOCK)

    def body(in_ref, out_ref):
        out_ref[...] = in_ref[...].reshape(8, RETILE_BLOCK // 128, 128)

    return pl.pallas_call(
        body,
        grid=grid,
        in_specs=[pl.BlockSpec((8, RETILE_BLOCK), lambda s, j: (s, j))],
        out_specs=pl.BlockSpec(
            (8, RETILE_BLOCK // 128, 128), lambda s, j: (s, j, 0)
        ),
        out_shape=jax.ShapeDtypeStruct((EMBED, VPAD // 128, 128), jnp.float32),
    )(tok_t)


F_CHUNK = 1600
F_NCHUNKS = VOCAB // F_CHUNK


def _make_fold_kernel():
    mesh = plsc.VectorSubcoreMesh(core_axis_name="c", subcore_axis_name="s")
    max_chunks = (F_NCHUNKS + NWORKERS - 1) // NWORKERS

    @functools.partial(
        pl.kernel,
        mesh=mesh,
        compiler_params=pltpu.CompilerParams(
            use_tc_tiling_on_sc=False, needs_layout_passes=False
        ),
        out_type=jax.ShapeDtypeStruct((VOCAB, EMBED), jnp.float32),
        scratch_types=[
            pltpu.VMEM((EMBED * F_CHUNK,), jnp.float32),
            pltpu.VMEM((F_CHUNK, EMBED), jnp.float32),
            pltpu.SemaphoreType.DMA,
        ],
    )
    def fold_kernel(src_hbm, out_hbm, inbuf, outbuf, sem):
        wid = lax.axis_index("s") * 2 + lax.axis_index("c")
        base_lo = lax.iota(jnp.int32, 16) * F_CHUNK
        base_hi = base_lo + 16 * F_CHUNK

        def g_body(k, carry):
            c = wid + NWORKERS * k

            @pl.when(c < F_NCHUNKS)
            def _():
                v0 = c * F_CHUNK
                cps = [
                    pltpu.async_copy(
                        src_hbm.at[pl.ds(d * VPAD + v0, F_CHUNK)],
                        inbuf.at[pl.ds(d * F_CHUNK, F_CHUNK)],
                        sem,
                    )
                    for d in range(EMBED)
                ]
                for cp in cps:
                    cp.wait()

                def i_body(i, carry2):
                    outbuf[i, pl.ds(0, 16)] = plsc.load_gather(
                        inbuf, [base_lo + i]
                    )
                    outbuf[i, pl.ds(16, 16)] = plsc.load_gather(
                        inbuf, [base_hi + i]
                    )
                    return carry2

                lax.fori_loop(0, F_CHUNK, i_body, 0)
                pltpu.sync_copy(outbuf, out_hbm.at[pl.ds(v0, F_CHUNK)])

            return carry

        lax.fori_loop(0, max_chunks, g_body, 0)

    return fold_kernel


OUT_BLOCK_B = 128


def _transpose_out(rows128, batch):
    per_b128 = MAXLEN * EMBED // 128
    grid = batch // OUT_BLOCK_B
    rows_in = OUT_BLOCK_B * per_b128

    def body(in_ref, out_ref):
        blk = in_ref[...].reshape(per_b128, OUT_BLOCK_B, 128)
        y = jnp.transpose(blk, (0, 2, 1))
        out_ref[...] = y.reshape(MAXLEN, EMBED, OUT_BLOCK_B)

    return pl.pallas_call(
        body,
        grid=(grid,),
        in_specs=[pl.BlockSpec((rows_in, 128), lambda i: (i, 0))],
        out_specs=pl.BlockSpec((MAXLEN, EMBED, OUT_BLOCK_B), lambda i: (0, 0, i)),
        out_shape=jax.ShapeDtypeStruct((MAXLEN, EMBED, batch), jnp.float32),
    )(rows128)


TSLAB = 8
BBLOCK = 128


@functools.cache
def _make_gather_kernel(n_rows: int):
    batch = n_rows // MAXLEN
    n_chunks = MAXLEN // TSLAB
    chunk_rows = TSLAB * BBLOCK
    stage_f = TSLAB * BBLOCK * EMBED
    wblock = BBLOCK * MAXLEN * EMBED

    mesh = plsc.VectorSubcoreMesh(core_axis_name="c", subcore_axis_name="s")

    @functools.partial(
        pl.kernel,
        mesh=mesh,
        compiler_params=pltpu.CompilerParams(use_tc_tiling_on_sc=False),
        out_type=jax.ShapeDtypeStruct((n_rows * EMBED,), jnp.float32),
        scratch_types=[
            pltpu.VMEM((TSLAB, BBLOCK), jnp.int32),
            pltpu.VMEM((chunk_rows, EMBED), jnp.float32),
            pltpu.VMEM((stage_f,), jnp.float32),
            pltpu.VMEM((MAXLEN, EMBED), jnp.float32),
            pltpu.SemaphoreType.DMA,
        ],
    )
    def emb_kernel(xt_hbm, tok_hbm, pos_hbm, out_hbm, idx_v, rows_v, stage_v,
                   pos_v, sem):
        wid = lax.axis_index("s") * 2 + lax.axis_index("c")
        pltpu.sync_copy(pos_hbm, pos_v)

        def chunk_body(g, carry):
            t0 = g * TSLAB
            idx_cps = [
                pltpu.async_copy(
                    xt_hbm.at[pl.ds((t0 + tl) * batch + BBLOCK * wid, BBLOCK)],
                    idx_v.at[tl],
                    sem,
                )
                for tl in range(TSLAB)
            ]
            for c in idx_cps:
                c.wait()
            copies = [
                pltpu.async_copy(
                    tok_hbm.at[idx_v.at[tl]],
                    rows_v.at[pl.ds(tl * BBLOCK, BBLOCK)],
                    sem,
                )
                for tl in range(TSLAB)
            ]
            for c in copies:
                c.wait()

            pvs = [
                (
                    pos_v[t0 + tl, pl.ds(0, LANES)],
                    pos_v[t0 + tl, pl.ds(LANES, LANES)],
                )
                for tl in range(TSLAB)
            ]

            def bb_body(bb, carry2):
                for tl in range(TSLAB):
                    r = tl * BBLOCK + bb
                    o = (tl // 4) * (BBLOCK * 128) + bb * 128 + (tl % 4) * EMBED
                    stage_v[pl.ds(o, LANES)] = (
                        rows_v[r, pl.ds(0, LANES)] + pvs[tl][0]
                    )
                    stage_v[pl.ds(o + LANES, LANES)] = (
                        rows_v[r, pl.ds(LANES, LANES)] + pvs[tl][1]
                    )
                return carry2

            lax.fori_loop(0, BBLOCK, bb_body, 0)

            pltpu.sync_copy(
                stage_v, out_hbm.at[pl.ds(wid * wblock + g * stage_f, stage_f)]
            )
            return carry

        lax.fori_loop(0, n_chunks, chunk_body, 0)

    return emb_kernel


def kernel(x, token_emb, pos_emb):
    batch, maxlen = x.shape
    n_rows = batch * maxlen
    xt = jnp.transpose(x).astype(jnp.int32).reshape(-1)
    tok_lin = _linearize_table(token_emb.T).reshape(VOCAB, EMBED)
    rows = _make_gather_kernel(n_rows)(xt, tok_lin, pos_emb)
    out_tdb = _transpose_out(rows.reshape(n_rows * EMBED // 128, 128), batch)
    return jnp.transpose(out_tdb, (2, 0, 1))


# speedup vs baseline: 1.0009x; 1.0004x over previous
import functools

import jax
import jax.numpy as jnp
from jax import lax
from jax.experimental import pallas as pl
from jax.experimental.pallas import tpu as pltpu
from jax.experimental.pallas import tpu_sc as plsc

MAXLEN = 200
EMBED = 32
LANES = 16
NWORKERS = 32
GATHER_ROWS = 100
GATHERS_PER_CHUNK = 16
CHUNK = GATHER_ROWS * GATHERS_PER_CHUNK

VOCAB = 1000000
LIN_BLOCK_V = 2048


def _linearize_table(tok_t):
    grid = (VOCAB + LIN_BLOCK_V - 1) // LIN_BLOCK_V
    rows_out = LIN_BLOCK_V // 4

    def body(in_ref, out_ref):
        x = in_ref[...]
        z = jnp.concatenate([x, x, x, x], axis=0)
        zt = z.T.reshape(rows_out, 4, 128)
        lane = lax.broadcasted_iota(jnp.int32, (rows_out, 128), 1)
        q = lane // EMBED
        out = zt[:, 0, :]
        for k in range(1, 4):
            out = jnp.where(q == k, zt[:, k, :], out)
        out_ref[...] = out

    return pl.pallas_call(
        body,
        grid=(grid,),
        in_specs=[pl.BlockSpec((EMBED, LIN_BLOCK_V), lambda i: (0, i))],
        out_specs=pl.BlockSpec((rows_out, 128), lambda i: (i, 0)),
        out_shape=jax.ShapeDtypeStruct((VOCAB * EMBED // 128, 128), jnp.float32),
    )(tok_t)


VPAD = 1000448
RETILE_BLOCK = 8192

def _retile_table(tok_t):
    grid = (EMBED // 8, (VOCAB + RETILE_BLOCK - 1) // RETILE_BLOCK)

    def body(in_ref, out_ref):
        out_ref[...] = in_ref[...].reshape(8, RETILE_BLOCK // 128, 128)

    return pl.pallas_call(
        body,
        grid=grid,
        in_specs=[pl.BlockSpec((8, RETILE_BLOCK), lambda s, j: (s, j))],
        out_specs=pl.BlockSpec(
            (8, RETILE_BLOCK // 128, 128), lambda s, j: (s, j, 0)
        ),
        out_shape=jax.ShapeDtypeStruct((EMBED, VPAD // 128, 128), jnp.float32),
    )(tok_t)


F_CHUNK = 1600
F_NCHUNKS = VOCAB // F_CHUNK


def _make_fold_kernel():
    mesh = plsc.VectorSubcoreMesh(core_axis_name="c", subcore_axis_name="s")
    max_chunks = (F_NCHUNKS + NWORKERS - 1) // NWORKERS

    @functools.partial(
        pl.kernel,
        mesh=mesh,
        compiler_params=pltpu.CompilerParams(
            use_tc_tiling_on_sc=False, needs_layout_passes=False
        ),
        out_type=jax.ShapeDtypeStruct((VOCAB, EMBED), jnp.float32),
        scratch_types=[
            pltpu.VMEM((EMBED * F_CHUNK,), jnp.float32),
            pltpu.VMEM((F_CHUNK, EMBED), jnp.float32),
            pltpu.SemaphoreType.DMA,
        ],
    )
    def fold_kernel(src_hbm, out_hbm, inbuf, outbuf, sem):
        wid = lax.axis_index("s") * 2 + lax.axis_index("c")
        base_lo = lax.iota(jnp.int32, 16) * F_CHUNK
        base_hi = base_lo + 16 * F_CHUNK

        def g_body(k, carry):
            c = wid + NWORKERS * k

            @pl.when(c < F_NCHUNKS)
            def _():
                v0 = c * F_CHUNK
                cps = [
                    pltpu.async_copy(
                        src_hbm.at[pl.ds(d * VPAD + v0, F_CHUNK)],
                        inbuf.at[pl.ds(d * F_CHUNK, F_CHUNK)],
                        sem,
                    )
                    for d in range(EMBED)
                ]
                for cp in cps:
                    cp.wait()

                def i_body(i, carry2):
                    outbuf[i, pl.ds(0, 16)] = plsc.load_gather(
                        inbuf, [base_lo + i]
                    )
                    outbuf[i, pl.ds(16, 16)] = plsc.load_gather(
                        inbuf, [base_hi + i]
                    )
                    return carry2

                lax.fori_loop(0, F_CHUNK, i_body, 0)
                pltpu.sync_copy(outbuf, out_hbm.at[pl.ds(v0, F_CHUNK)])

            return carry

        lax.fori_loop(0, max_chunks, g_body, 0)

    return fold_kernel


OUT_BLOCK_B = 128


def _transpose_out(rows128, batch):
    per_b128 = MAXLEN * EMBED // 128
    grid = batch // OUT_BLOCK_B
    rows_in = OUT_BLOCK_B * per_b128

    def body(in_ref, out_ref):
        blk = in_ref[...].reshape(per_b128, OUT_BLOCK_B, 128)
        y = jnp.transpose(blk, (0, 2, 1))
        out_ref[...] = y.reshape(MAXLEN, EMBED, OUT_BLOCK_B)

    return pl.pallas_call(
        body,
        grid=(grid,),
        in_specs=[pl.BlockSpec((rows_in, 128), lambda i: (i, 0))],
        out_specs=pl.BlockSpec((MAXLEN, EMBED, OUT_BLOCK_B), lambda i: (0, 0, i)),
        out_shape=jax.ShapeDtypeStruct((MAXLEN, EMBED, batch), jnp.float32),
    )(rows128)


TSLAB = 8
BBLOCK = 128


@functools.cache
def _make_gather_kernel(n_rows: int):
    batch = n_rows // MAXLEN
    n_chunks = MAXLEN // TSLAB
    chunk_rows = TSLAB * BBLOCK
    stage_f = TSLAB * BBLOCK * EMBED
    wblock = BBLOCK * MAXLEN * EMBED

    mesh = plsc.VectorSubcoreMesh(core_axis_name="c", subcore_axis_name="s")

    @functools.partial(
        pl.kernel,
        mesh=mesh,
        compiler_params=pltpu.CompilerParams(use_tc_tiling_on_sc=False),
        out_type=jax.ShapeDtypeStruct((n_rows * EMBED // 128, 128), jnp.float32),
        scratch_types=[
            pltpu.VMEM((TSLAB, BBLOCK), jnp.int32),
            pltpu.VMEM((chunk_rows, EMBED), jnp.float32),
            pltpu.VMEM((stage_f // 128, 128), jnp.float32),
            pltpu.VMEM((MAXLEN, EMBED), jnp.float32),
            pltpu.SemaphoreType.DMA,
        ],
    )
    def emb_kernel(xt_hbm, tok_hbm, pos_hbm, out_hbm, idx_v, rows_v, stage_v,
                   pos_v, sem):
        wid = lax.axis_index("s") * 2 + lax.axis_index("c")
        pltpu.sync_copy(pos_hbm, pos_v)

        def chunk_body(g, carry):
            t0 = g * TSLAB
            idx_cps = [
                pltpu.async_copy(
                    xt_hbm.at[pl.ds((t0 + tl) * batch + BBLOCK * wid, BBLOCK)],
                    idx_v.at[tl],
                    sem,
                )
                for tl in range(TSLAB)
            ]
            for c in idx_cps:
                c.wait()
            copies = [
                pltpu.async_copy(
                    tok_hbm.at[idx_v.at[tl]],
                    rows_v.at[pl.ds(tl * BBLOCK, BBLOCK)],
                    sem,
                )
                for tl in range(TSLAB)
            ]
            for c in copies:
                c.wait()

            pvs = [
                (
                    pos_v[t0 + tl, pl.ds(0, LANES)],
                    pos_v[t0 + tl, pl.ds(LANES, LANES)],
                )
                for tl in range(TSLAB)
            ]

            def bb_body(bb, carry2):
                for tl in range(TSLAB):
                    r = tl * BBLOCK + bb
                    so = (tl // 4) * BBLOCK + bb
                    sl = (tl % 4) * EMBED
                    stage_v[so, pl.ds(sl, LANES)] = (
                        rows_v[r, pl.ds(0, LANES)] + pvs[tl][0]
                    )
                    stage_v[so, pl.ds(sl + LANES, LANES)] = (
                        rows_v[r, pl.ds(LANES, LANES)] + pvs[tl][1]
                    )
                return carry2

            lax.fori_loop(0, BBLOCK, bb_body, 0)

            row0 = (wid * wblock + g * stage_f) // 128
            pltpu.sync_copy(
                stage_v, out_hbm.at[pl.ds(row0, stage_f // 128)]
            )
            return carry

        lax.fori_loop(0, n_chunks, chunk_body, 0)

    return emb_kernel


def kernel(x, token_emb, pos_emb):
    batch, maxlen = x.shape
    n_rows = batch * maxlen
    xt = jnp.transpose(x).astype(jnp.int32).reshape(-1)
    tok_lin = _linearize_table(token_emb.T).reshape(VOCAB, EMBED)
    rows = _make_gather_kernel(n_rows)(xt, tok_lin, pos_emb)
    out_tdb = _transpose_out(rows, batch)
    return jnp.transpose(out_tdb, (2, 0, 1))


# speedup vs baseline: 1.1586x; 1.1576x over previous
import functools

import jax
import jax.numpy as jnp
from jax import lax
from jax.experimental import pallas as pl
from jax.experimental.pallas import tpu as pltpu
from jax.experimental.pallas import tpu_sc as plsc

MAXLEN = 200
EMBED = 32
LANES = 16
NWORKERS = 32
GATHER_ROWS = 100
GATHERS_PER_CHUNK = 16
CHUNK = GATHER_ROWS * GATHERS_PER_CHUNK

VOCAB = 1000000
LIN_BLOCK_V = 2048


def _linearize_table(tok_t):
    grid = (VOCAB + LIN_BLOCK_V - 1) // LIN_BLOCK_V
    rows_out = LIN_BLOCK_V // 4

    def body(in_ref, out_ref):
        x = in_ref[...]
        z = jnp.concatenate([x, x, x, x], axis=0)
        zt = z.T.reshape(rows_out, 4, 128)
        lane = lax.broadcasted_iota(jnp.int32, (rows_out, 128), 1)
        q = lane // EMBED
        out = zt[:, 0, :]
        for k in range(1, 4):
            out = jnp.where(q == k, zt[:, k, :], out)
        out_ref[...] = out

    return pl.pallas_call(
        body,
        grid=(grid,),
        in_specs=[pl.BlockSpec((EMBED, LIN_BLOCK_V), lambda i: (0, i))],
        out_specs=pl.BlockSpec((rows_out, 128), lambda i: (i, 0)),
        out_shape=jax.ShapeDtypeStruct((VOCAB * EMBED // 128, 128), jnp.float32),
    )(tok_t)


OUT_BLOCK_B = 128


def _transpose_out(rows128, batch):
    per_b128 = MAXLEN * EMBED // 128
    grid = batch // OUT_BLOCK_B
    rows_in = OUT_BLOCK_B * per_b128

    def body(in_ref, out_ref):
        blk = in_ref[...].reshape(OUT_BLOCK_B, per_b128, 128)
        y = jnp.transpose(blk, (1, 2, 0))
        out_ref[...] = y.reshape(MAXLEN, EMBED, OUT_BLOCK_B)

    return pl.pallas_call(
        body,
        grid=(grid,),
        in_specs=[pl.BlockSpec((rows_in, 128), lambda i: (i, 0))],
        out_specs=pl.BlockSpec((MAXLEN, EMBED, OUT_BLOCK_B), lambda i: (0, 0, i)),
        out_shape=jax.ShapeDtypeStruct((MAXLEN, EMBED, batch), jnp.float32),
    )(rows128)


@functools.cache
def _make_gather_kernel(n_rows: int):
    rows_per_w = n_rows // NWORKERS
    n_chunks = rows_per_w // CHUNK
    idx_rows_per_w = rows_per_w // GATHER_ROWS
    cycles = CHUNK // MAXLEN

    mesh = plsc.VectorSubcoreMesh(core_axis_name="c", subcore_axis_name="s")

    @functools.partial(
        pl.kernel,
        mesh=mesh,
        compiler_params=pltpu.CompilerParams(use_tc_tiling_on_sc=False),
        out_type=jax.ShapeDtypeStruct((n_rows, EMBED), jnp.float32),
        scratch_types=[
            pltpu.VMEM((GATHERS_PER_CHUNK, GATHER_ROWS), jnp.int32),
            pltpu.VMEM((CHUNK, EMBED), jnp.float32),
            pltpu.VMEM((MAXLEN, EMBED), jnp.float32),
            pltpu.SemaphoreType.DMA,
        ],
    )
    def emb_kernel(x_hbm, tok_hbm, pos_hbm, out_hbm, idx_v, rows_v, pos_v, sem):
        wid = lax.axis_index("s") * 2 + lax.axis_index("c")
        pltpu.sync_copy(pos_hbm, pos_v)

        def chunk_body(g, carry):
            r0 = wid * idx_rows_per_w + g * GATHERS_PER_CHUNK
            pltpu.sync_copy(x_hbm.at[pl.ds(r0, GATHERS_PER_CHUNK)], idx_v)
            copies = [
                pltpu.async_copy(
                    tok_hbm.at[idx_v.at[j]],
                    rows_v.at[pl.ds(j * GATHER_ROWS, GATHER_ROWS)],
                    sem,
                )
                for j in range(GATHERS_PER_CHUNK)
            ]
            for c in copies:
                c.wait()

            def t_body(t, carry2):
                p0 = pos_v[t, pl.ds(0, LANES)]
                p1 = pos_v[t, pl.ds(LANES, LANES)]
                for cyc in range(cycles):
                    r = cyc * MAXLEN + t
                    rows_v[r, pl.ds(0, LANES)] = rows_v[r, pl.ds(0, LANES)] + p0
                    rows_v[r, pl.ds(LANES, LANES)] = (
                        rows_v[r, pl.ds(LANES, LANES)] + p1
                    )
                return carry2

            lax.fori_loop(0, MAXLEN, t_body, 0)

            base = wid * rows_per_w + g * CHUNK
            pltpu.sync_copy(rows_v, out_hbm.at[pl.ds(base, CHUNK)])
            return carry

        lax.fori_loop(0, n_chunks, chunk_body, 0)

    return emb_kernel


def kernel(x, token_emb, pos_emb):
    batch, maxlen = x.shape
    n_rows = batch * maxlen
    x2 = x.reshape(n_rows // GATHER_ROWS, GATHER_ROWS).astype(jnp.int32)
    tok_lin = _linearize_table(token_emb.T).reshape(VOCAB, EMBED)
    rows = _make_gather_kernel(n_rows)(x2, tok_lin, pos_emb)
    out_tdb = _transpose_out(rows.reshape(n_rows * EMBED // 128, 128), batch)
    return jnp.transpose(out_tdb, (2, 0, 1))
